# baseline (speedup 1.0000x reference)
import functools

import jax
import jax.numpy as jnp
from jax import lax
from jax.experimental import pallas as pl
from jax.experimental.pallas import tpu as pltpu
from jax.experimental.pallas import tpu_sc as plsc

N = 10000
E = 160000
F = 256
FH = F // 2
NC, NS = 2, 16
CH = 125
ROWS = E // CH
RPT_DEG = ROWS // (NC * NS)
RPT_AGG = ROWS // NS

_mesh = plsc.VectorSubcoreMesh(core_axis_name="c", subcore_axis_name="s")


@functools.partial(
    pl.kernel,
    out_type=jax.ShapeDtypeStruct((NC * N,), jnp.float32),
    mesh=_mesh,
    scratch_types=[
        pltpu.VMEM((RPT_DEG, CH), jnp.int32),
        pltpu.VMEM((128,), jnp.float32),
        pltpu.VMEM((640,), jnp.float32),
        pltpu.VMEM_SHARED((N,), jnp.float32),
    ],
)
def _deg(dst_hbm, out_hbm, didx_v, ones_v, buf_v, deg_sh):
    c = lax.axis_index("c")
    s = lax.axis_index("s")
    wid = c * NS + s

    for i in range(40):
        buf_v[pl.ds(16 * i, 16)] = jnp.zeros((16,), jnp.float32)

    @pl.when(s < NS - 1)
    def _():
        pltpu.sync_copy(buf_v.at[pl.ds(0, 640)], deg_sh.at[pl.ds(s * 640, 640)])

    @pl.when(s == NS - 1)
    def _():
        pltpu.sync_copy(buf_v.at[pl.ds(0, 400)], deg_sh.at[pl.ds(s * 640, 400)])

    for i in range(8):
        ones_v[pl.ds(16 * i, 16)] = jnp.ones((16,), jnp.float32)

    pltpu.sync_copy(dst_hbm.at[pl.ds(wid * RPT_DEG, RPT_DEG)], didx_v)
    plsc.subcore_barrier()

    def body(j, _):
        pltpu.sync_copy(ones_v.at[pl.ds(0, CH)], deg_sh.at[didx_v.at[j]], add=True)
        return 0

    lax.fori_loop(0, RPT_DEG, body, 0)
    plsc.subcore_barrier()

    @pl.when(s < NS - 1)
    def _():
        pltpu.sync_copy(deg_sh.at[pl.ds(s * 640, 640)], buf_v.at[pl.ds(0, 640)])
        pltpu.sync_copy(
            buf_v.at[pl.ds(0, 640)], out_hbm.at[pl.ds(c * N + s * 640, 640)]
        )

    @pl.when(s == NS - 1)
    def _():
        pltpu.sync_copy(deg_sh.at[pl.ds(s * 640, 400)], buf_v.at[pl.ds(0, 400)])
        pltpu.sync_copy(
            buf_v.at[pl.ds(0, 400)], out_hbm.at[pl.ds(c * N + s * 640, 400)]
        )


@functools.partial(
    pl.kernel,
    out_type=jax.ShapeDtypeStruct((NC * N, FH), jnp.float32),
    mesh=_mesh,
    scratch_types=[
        pltpu.VMEM((RPT_AGG // 2, CH), jnp.int32),
        pltpu.VMEM((RPT_AGG // 2, CH), jnp.int32),
        pltpu.VMEM((128, FH), jnp.float32),
        pltpu.VMEM((CH, FH), jnp.float32),
        pltpu.VMEM((5, 128), jnp.int32),
        pltpu.VMEM((16,), jnp.int32),
        pltpu.VMEM_SHARED((N, FH), jnp.float32),
        pltpu.SemaphoreType.DMA,
        pltpu.SemaphoreType.DMA,
    ],
)
def _agg(hw_hbm, src_hbm, dst_hbm, out_hbm, sidx_v, didx_v, buf_v, bufb_v,
         zidx_v, tidx_v, acc_sh, sema, semb):
    c = lax.axis_index("c")
    s = lax.axis_index("s")

    for j in range(5):
        for k in range(8):
            zidx_v[j, pl.ds(16 * k, 16)] = (
                s * 640 + 128 * j + 16 * k + lax.iota(jnp.int32, 16)
            )
    tidx_v[...] = s * 640 + 384 + lax.iota(jnp.int32, 16)

    def zbody(i, _):
        for k in range(FH // 16):
            buf_v[i, pl.ds(16 * k, 16)] = jnp.zeros((16,), jnp.float32)
        return 0

    lax.fori_loop(0, 128, zbody, 0)

    @pl.when(s < NS - 1)
    def _():
        for j in range(5):
            pltpu.sync_copy(buf_v, acc_sh.at[zidx_v.at[j]])

    @pl.when(s == NS - 1)
    def _():
        for j in range(3):
            pltpu.sync_copy(buf_v, acc_sh.at[zidx_v.at[j]])
        pltpu.sync_copy(buf_v.at[pl.ds(0, 16)], acc_sh.at[tidx_v])

    plsc.subcore_barrier()

    HR = RPT_AGG // 2
    bufa = buf_v.at[pl.ds(0, CH)]
    hw_c = hw_hbm.at[pl.ds(pl.multiple_of(c * N, 8), N)]
    for h in range(2):
        pltpu.sync_copy(
            src_hbm.at[pl.ds(s * RPT_AGG + h * HR, HR)], sidx_v
        )
        pltpu.sync_copy(
            dst_hbm.at[pl.ds(s * RPT_AGG + h * HR, HR)], didx_v
        )
        pltpu.async_copy(hw_c.at[sidx_v.at[0]], bufa, sema)

        def body(t, _):
            j0 = 2 * t
            db = pltpu.async_copy(hw_c.at[sidx_v.at[j0 + 1]], bufb_v, semb)
            pltpu.make_async_copy(hw_c.at[sidx_v.at[j0]], bufa, sema).wait()
            pltpu.sync_copy(bufa, acc_sh.at[didx_v.at[j0]], add=True)

            @pl.when(t < HR // 2 - 1)
            def _():
                pltpu.async_copy(hw_c.at[sidx_v.at[j0 + 2]], bufa, sema)

            db.wait()
            pltpu.sync_copy(bufb_v, acc_sh.at[didx_v.at[j0 + 1]], add=True)
            return 0

        lax.fori_loop(0, HR // 2, body, 0)
    plsc.subcore_barrier()

    @pl.when(s < NS - 1)
    def _():
        for j in range(5):
            pltpu.async_copy(acc_sh.at[zidx_v.at[j]], buf_v, sema).wait()
            pltpu.sync_copy(
                buf_v, out_hbm.at[pl.ds(c * N + s * 640 + 128 * j, 128)]
            )

    @pl.when(s == NS - 1)
    def _():
        for j in range(3):
            pltpu.async_copy(acc_sh.at[zidx_v.at[j]], buf_v, sema).wait()
            pltpu.sync_copy(
                buf_v, out_hbm.at[pl.ds(c * N + s * 640 + 128 * j, 128)]
            )
        pltpu.async_copy(acc_sh.at[tidx_v], buf_v.at[pl.ds(0, 16)], sema).wait()
        pltpu.sync_copy(
            buf_v.at[pl.ds(0, 16)], out_hbm.at[pl.ds(c * N + s * 640 + 384, 16)]
        )


def _mm_body(h_ref, w_ref, degp_ref, out_ref):
    deg = degp_ref[:, 0] + degp_ref[:, 1]
    norm = lax.rsqrt(jnp.where(deg > 0.0, deg, 1.0))
    acc = jnp.dot(h_ref[...], w_ref[...], preferred_element_type=jnp.float32)
    out_ref[...] = acc * norm[:, None]


def _matmul(h, W, degp):
    bm = 2000
    grid = (N // bm, NC)
    return pl.pallas_call(
        _mm_body,
        grid=grid,
        in_specs=[
            pl.BlockSpec((bm, F), lambda i, c: (i, 0)),
            pl.BlockSpec((F, FH), lambda i, c: (0, c)),
            pl.BlockSpec((bm, NC), lambda i, c: (i, 0)),
        ],
        out_specs=pl.BlockSpec((bm, FH), lambda i, c: (c * (N // bm) + i, 0)),
        out_shape=jax.ShapeDtypeStruct((NC * N, FH), jnp.float32),
    )(h, W, degp)


def _final_body(agg_ref, degp_ref, b_ref, out_ref):
    deg = degp_ref[:, 0] + degp_ref[:, 1]
    norm = lax.rsqrt(jnp.where(deg > 0.0, deg, 1.0))
    brow = jnp.where(pl.program_id(1) == 0, b_ref[0, :], b_ref[1, :])
    out_ref[...] = jnp.maximum(agg_ref[...] * norm[:, None] + brow, 0.0)


def _final(agg, degp, b2):
    bm = 1000
    nb = N // bm
    grid = (nb, NC)
    return pl.pallas_call(
        _final_body,
        grid=grid,
        in_specs=[
            pl.BlockSpec((bm, FH), lambda i, c: (c * nb + i, 0)),
            pl.BlockSpec((bm, NC), lambda i, c: (i, 0)),
            pl.BlockSpec((NC, FH), lambda i, c: (0, 0)),
        ],
        out_specs=pl.BlockSpec((bm, FH), lambda i, c: (i, c)),
        out_shape=jax.ShapeDtypeStruct((N, F), jnp.float32),
    )(agg, degp, b2)


def kernel(h, edge_index, W, b):
    ei = edge_index.astype(jnp.int32)
    src = ei[0]
    dst = ei[1]
    src2 = src.reshape(ROWS, CH)
    dst2 = dst.reshape(ROWS, CH)

    degp = _deg(dst2).reshape(NC, N).T
    hw = _matmul(h, W, degp)
    agg = _agg(hw, src2, dst2)
    return _final(agg, degp, b.reshape(NC, FH))


# speedup vs baseline: 1.0189x; 1.0189x over previous
import functools

import jax
import jax.numpy as jnp
from jax import lax
from jax.experimental import pallas as pl
from jax.experimental.pallas import tpu as pltpu
from jax.experimental.pallas import tpu_sc as plsc

N = 10000
E = 160000
F = 256
FH = F // 2
NC, NS = 2, 16
CH = 125
ROWS = E // CH
RPT_DEG = ROWS // (NC * NS)
RPT_AGG = ROWS // NS

_mesh = plsc.VectorSubcoreMesh(core_axis_name="c", subcore_axis_name="s")


@functools.partial(
    pl.kernel,
    out_type=jax.ShapeDtypeStruct((NC * N,), jnp.float32),
    mesh=_mesh,
    scratch_types=[
        pltpu.VMEM((RPT_DEG, CH), jnp.int32),
        pltpu.VMEM((128,), jnp.float32),
        pltpu.VMEM((640,), jnp.float32),
        pltpu.VMEM_SHARED((N,), jnp.float32),
    ],
)
def _deg(dst_hbm, out_hbm, didx_v, ones_v, buf_v, deg_sh):
    c = lax.axis_index("c")
    s = lax.axis_index("s")
    wid = c * NS + s

    for i in range(40):
        buf_v[pl.ds(16 * i, 16)] = jnp.zeros((16,), jnp.float32)

    @pl.when(s < NS - 1)
    def _():
        pltpu.sync_copy(buf_v.at[pl.ds(0, 640)], deg_sh.at[pl.ds(s * 640, 640)])

    @pl.when(s == NS - 1)
    def _():
        pltpu.sync_copy(buf_v.at[pl.ds(0, 400)], deg_sh.at[pl.ds(s * 640, 400)])

    for i in range(8):
        ones_v[pl.ds(16 * i, 16)] = jnp.ones((16,), jnp.float32)

    pltpu.sync_copy(dst_hbm.at[pl.ds(wid * RPT_DEG, RPT_DEG)], didx_v)
    plsc.subcore_barrier()

    def body(j, _):
        pltpu.sync_copy(ones_v.at[pl.ds(0, CH)], deg_sh.at[didx_v.at[j]], add=True)
        return 0

    lax.fori_loop(0, RPT_DEG, body, 0)
    plsc.subcore_barrier()

    @pl.when(s < NS - 1)
    def _():
        pltpu.sync_copy(deg_sh.at[pl.ds(s * 640, 640)], buf_v.at[pl.ds(0, 640)])
        pltpu.sync_copy(
            buf_v.at[pl.ds(0, 640)], out_hbm.at[pl.ds(c * N + s * 640, 640)]
        )

    @pl.when(s == NS - 1)
    def _():
        pltpu.sync_copy(deg_sh.at[pl.ds(s * 640, 400)], buf_v.at[pl.ds(0, 400)])
        pltpu.sync_copy(
            buf_v.at[pl.ds(0, 400)], out_hbm.at[pl.ds(c * N + s * 640, 400)]
        )


@functools.partial(
    pl.kernel,
    out_type=jax.ShapeDtypeStruct((NC * N, FH), jnp.float32),
    mesh=_mesh,
    scratch_types=[
        pltpu.VMEM((RPT_AGG // 2, CH), jnp.int32),
        pltpu.VMEM((RPT_AGG // 2, CH), jnp.int32),
        pltpu.VMEM((128, FH), jnp.float32),
        pltpu.VMEM((CH, FH), jnp.float32),
        pltpu.VMEM((5, 128), jnp.int32),
        pltpu.VMEM((16,), jnp.int32),
        pltpu.VMEM_SHARED((N, FH), jnp.float32),
        pltpu.SemaphoreType.DMA,
        pltpu.SemaphoreType.DMA,
    ],
)
def _agg(hw_hbm, src_hbm, dst_hbm, out_hbm, sidx_v, didx_v, buf_v, bufb_v,
         zidx_v, tidx_v, acc_sh, sema, semb):
    c = lax.axis_index("c")
    s = lax.axis_index("s")

    for j in range(5):
        for k in range(8):
            zidx_v[j, pl.ds(16 * k, 16)] = (
                s * 640 + 128 * j + 16 * k + lax.iota(jnp.int32, 16)
            )
    tidx_v[...] = s * 640 + 384 + lax.iota(jnp.int32, 16)

    def zbody(i, _):
        for k in range(FH // 16):
            buf_v[i, pl.ds(16 * k, 16)] = jnp.zeros((16,), jnp.float32)
        return 0

    lax.fori_loop(0, 128, zbody, 0)

    @pl.when(s < NS - 1)
    def _():
        for j in range(5):
            pltpu.sync_copy(buf_v, acc_sh.at[zidx_v.at[j]])

    @pl.when(s == NS - 1)
    def _():
        for j in range(3):
            pltpu.sync_copy(buf_v, acc_sh.at[zidx_v.at[j]])
        pltpu.sync_copy(buf_v.at[pl.ds(0, 16)], acc_sh.at[tidx_v])

    plsc.subcore_barrier()

    HR = RPT_AGG // 2
    bufa = buf_v.at[pl.ds(0, CH)]
    hw_c = hw_hbm.at[pl.ds(pl.multiple_of(c * N, 8), N)]
    for h in range(2):
        pltpu.sync_copy(
            src_hbm.at[pl.ds(s * RPT_AGG + h * HR, HR)], sidx_v
        )
        pltpu.sync_copy(
            dst_hbm.at[pl.ds(s * RPT_AGG + h * HR, HR)], didx_v
        )
        pltpu.async_copy(hw_c.at[sidx_v.at[0]], bufa, sema)

        def body(t, _):
            j0 = 2 * t
            db = pltpu.async_copy(hw_c.at[sidx_v.at[j0 + 1]], bufb_v, semb)
            pltpu.make_async_copy(hw_c.at[sidx_v.at[j0]], bufa, sema).wait()
            pltpu.sync_copy(bufa, acc_sh.at[didx_v.at[j0]], add=True)

            @pl.when(t < HR // 2 - 1)
            def _():
                pltpu.async_copy(hw_c.at[sidx_v.at[j0 + 2]], bufa, sema)

            db.wait()
            pltpu.sync_copy(bufb_v, acc_sh.at[didx_v.at[j0 + 1]], add=True)
            return 0

        lax.fori_loop(0, HR // 2, body, 0)
    plsc.subcore_barrier()

    @pl.when(s < NS - 1)
    def _():
        for j in range(5):
            pltpu.async_copy(acc_sh.at[zidx_v.at[j]], buf_v, sema).wait()
            pltpu.sync_copy(
                buf_v, out_hbm.at[pl.ds(c * N + s * 640 + 128 * j, 128)]
            )

    @pl.when(s == NS - 1)
    def _():
        for j in range(3):
            pltpu.async_copy(acc_sh.at[zidx_v.at[j]], buf_v, sema).wait()
            pltpu.sync_copy(
                buf_v, out_hbm.at[pl.ds(c * N + s * 640 + 128 * j, 128)]
            )
        pltpu.async_copy(acc_sh.at[tidx_v], buf_v.at[pl.ds(0, 16)], sema).wait()
        pltpu.sync_copy(
            buf_v.at[pl.ds(0, 16)], out_hbm.at[pl.ds(c * N + s * 640 + 384, 16)]
        )


def _mm_body(h_ref, w_ref, degp_ref, out_ref):
    deg = degp_ref[:, 0] + degp_ref[:, 1]
    norm = lax.rsqrt(jnp.where(deg > 0.0, deg, 1.0))
    acc = jnp.dot(h_ref[...], w_ref[...], preferred_element_type=jnp.float32)
    out_ref[...] = acc * norm[:, None]


def _matmul(h, W, degp):
    bm = 5000
    grid = (N // bm, NC)
    return pl.pallas_call(
        _mm_body,
        grid=grid,
        in_specs=[
            pl.BlockSpec((bm, F), lambda i, c: (i, 0)),
            pl.BlockSpec((F, FH), lambda i, c: (0, c)),
            pl.BlockSpec((bm, NC), lambda i, c: (i, 0)),
        ],
        out_specs=pl.BlockSpec((bm, FH), lambda i, c: (c * (N // bm) + i, 0)),
        out_shape=jax.ShapeDtypeStruct((NC * N, FH), jnp.float32),
    )(h, W, degp)


def _final_body(agg_ref, degp_ref, b_ref, out_ref):
    deg = degp_ref[:, 0] + degp_ref[:, 1]
    norm = lax.rsqrt(jnp.where(deg > 0.0, deg, 1.0))
    brow = jnp.where(pl.program_id(1) == 0, b_ref[0, :], b_ref[1, :])
    out_ref[...] = jnp.maximum(agg_ref[...] * norm[:, None] + brow, 0.0)


def _final(agg, degp, b2):
    bm = 1000
    nb = N // bm
    grid = (nb, NC)
    return pl.pallas_call(
        _final_body,
        grid=grid,
        in_specs=[
            pl.BlockSpec((bm, FH), lambda i, c: (c * nb + i, 0)),
            pl.BlockSpec((bm, NC), lambda i, c: (i, 0)),
            pl.BlockSpec((NC, FH), lambda i, c: (0, 0)),
        ],
        out_specs=pl.BlockSpec((bm, FH), lambda i, c: (i, c)),
        out_shape=jax.ShapeDtypeStruct((N, F), jnp.float32),
    )(agg, degp, b2)


def kernel(h, edge_index, W, b):
    ei = edge_index.astype(jnp.int32)
    src = ei[0]
    dst = ei[1]
    src2 = src.reshape(ROWS, CH)
    dst2 = dst.reshape(ROWS, CH)

    degp = _deg(dst2).reshape(NC, N).T
    hw = _matmul(h, W, degp)
    agg = _agg(hw, src2, dst2)
    return _final(agg, degp, b.reshape(NC, FH))


# speedup vs baseline: 1.0215x; 1.0026x over previous
import functools

import jax
import jax.numpy as jnp
from jax import lax
from jax.experimental import pallas as pl
from jax.experimental.pallas import tpu as pltpu
from jax.experimental.pallas import tpu_sc as plsc

N = 10000
E = 160000
F = 256
FH = F // 2
NC, NS = 2, 16
CH = 125
ROWS = E // CH
RPT_DEG = ROWS // (NC * NS)
RPT_AGG = ROWS // NS

_mesh = plsc.VectorSubcoreMesh(core_axis_name="c", subcore_axis_name="s")


@functools.partial(
    pl.kernel,
    out_type=jax.ShapeDtypeStruct((NC * N,), jnp.float32),
    mesh=_mesh,
    scratch_types=[
        pltpu.VMEM((RPT_DEG, CH), jnp.int32),
        pltpu.VMEM((128,), jnp.float32),
        pltpu.VMEM((640,), jnp.float32),
        pltpu.VMEM_SHARED((N,), jnp.float32),
    ],
)
def _deg(dst_hbm, out_hbm, didx_v, ones_v, buf_v, deg_sh):
    c = lax.axis_index("c")
    s = lax.axis_index("s")
    wid = c * NS + s

    for i in range(40):
        buf_v[pl.ds(16 * i, 16)] = jnp.zeros((16,), jnp.float32)

    @pl.when(s < NS - 1)
    def _():
        pltpu.sync_copy(buf_v.at[pl.ds(0, 640)], deg_sh.at[pl.ds(s * 640, 640)])

    @pl.when(s == NS - 1)
    def _():
        pltpu.sync_copy(buf_v.at[pl.ds(0, 400)], deg_sh.at[pl.ds(s * 640, 400)])

    for i in range(8):
        ones_v[pl.ds(16 * i, 16)] = jnp.ones((16,), jnp.float32)

    pltpu.sync_copy(dst_hbm.at[pl.ds(wid * RPT_DEG, RPT_DEG)], didx_v)
    plsc.subcore_barrier()

    def body(j, _):
        pltpu.sync_copy(ones_v.at[pl.ds(0, CH)], deg_sh.at[didx_v.at[j]], add=True)
        return 0

    lax.fori_loop(0, RPT_DEG, body, 0)
    plsc.subcore_barrier()

    @pl.when(s < NS - 1)
    def _():
        pltpu.sync_copy(deg_sh.at[pl.ds(s * 640, 640)], buf_v.at[pl.ds(0, 640)])
        pltpu.sync_copy(
            buf_v.at[pl.ds(0, 640)], out_hbm.at[pl.ds(c * N + s * 640, 640)]
        )

    @pl.when(s == NS - 1)
    def _():
        pltpu.sync_copy(deg_sh.at[pl.ds(s * 640, 400)], buf_v.at[pl.ds(0, 400)])
        pltpu.sync_copy(
            buf_v.at[pl.ds(0, 400)], out_hbm.at[pl.ds(c * N + s * 640, 400)]
        )


@functools.partial(
    pl.kernel,
    out_type=jax.ShapeDtypeStruct((NC * N, FH), jnp.float32),
    mesh=_mesh,
    scratch_types=[
        pltpu.VMEM((RPT_AGG // 2, CH), jnp.int32),
        pltpu.VMEM((RPT_AGG // 2, CH), jnp.int32),
        pltpu.VMEM((128, FH), jnp.float32),
        pltpu.VMEM((CH, FH), jnp.float32),
        pltpu.VMEM((5, 128), jnp.int32),
        pltpu.VMEM((16,), jnp.int32),
        pltpu.VMEM_SHARED((N, FH), jnp.float32),
        pltpu.SemaphoreType.DMA,
        pltpu.SemaphoreType.DMA,
    ],
)
def _agg(hw_hbm, src_hbm, dst_hbm, out_hbm, sidx_v, didx_v, buf_v, bufb_v,
         zidx_v, tidx_v, acc_sh, sema, semb):
    c = lax.axis_index("c")
    s = lax.axis_index("s")

    for j in range(5):
        for k in range(8):
            zidx_v[j, pl.ds(16 * k, 16)] = (
                s * 640 + 128 * j + 16 * k + lax.iota(jnp.int32, 16)
            )
    tidx_v[...] = s * 640 + 384 + lax.iota(jnp.int32, 16)

    def zbody(i, _):
        for k in range(FH // 16):
            buf_v[i, pl.ds(16 * k, 16)] = jnp.zeros((16,), jnp.float32)
        return 0

    lax.fori_loop(0, 128, zbody, 0)

    @pl.when(s < NS - 1)
    def _():
        for j in range(5):
            pltpu.sync_copy(buf_v, acc_sh.at[zidx_v.at[j]])

    @pl.when(s == NS - 1)
    def _():
        for j in range(3):
            pltpu.sync_copy(buf_v, acc_sh.at[zidx_v.at[j]])
        pltpu.sync_copy(buf_v.at[pl.ds(0, 16)], acc_sh.at[tidx_v])

    plsc.subcore_barrier()

    HR = RPT_AGG // 2
    bufa = buf_v.at[pl.ds(0, CH)]
    hw_c = hw_hbm.at[pl.ds(pl.multiple_of(c * N, 8), N)]
    for h in range(2):
        pltpu.sync_copy(
            src_hbm.at[pl.ds(s * RPT_AGG + h * HR, HR)], sidx_v
        )
        pltpu.sync_copy(
            dst_hbm.at[pl.ds(s * RPT_AGG + h * HR, HR)], didx_v
        )
        pltpu.async_copy(hw_c.at[sidx_v.at[0]], bufa, sema)

        def body(t, _):
            j0 = 2 * t
            db = pltpu.async_copy(hw_c.at[sidx_v.at[j0 + 1]], bufb_v, semb)
            pltpu.make_async_copy(hw_c.at[sidx_v.at[j0]], bufa, sema).wait()
            pltpu.sync_copy(bufa, acc_sh.at[didx_v.at[j0]], add=True)

            @pl.when(t < HR // 2 - 1)
            def _():
                pltpu.async_copy(hw_c.at[sidx_v.at[j0 + 2]], bufa, sema)

            db.wait()
            pltpu.sync_copy(bufb_v, acc_sh.at[didx_v.at[j0 + 1]], add=True)
            return 0

        lax.fori_loop(0, HR // 2, body, 0)
    plsc.subcore_barrier()

    @pl.when(s < NS - 1)
    def _():
        for j in range(5):
            pltpu.async_copy(acc_sh.at[zidx_v.at[j]], buf_v, sema).wait()
            pltpu.sync_copy(
                buf_v, out_hbm.at[pl.ds(c * N + s * 640 + 128 * j, 128)]
            )

    @pl.when(s == NS - 1)
    def _():
        for j in range(3):
            pltpu.async_copy(acc_sh.at[zidx_v.at[j]], buf_v, sema).wait()
            pltpu.sync_copy(
                buf_v, out_hbm.at[pl.ds(c * N + s * 640 + 128 * j, 128)]
            )
        pltpu.async_copy(acc_sh.at[tidx_v], buf_v.at[pl.ds(0, 16)], sema).wait()
        pltpu.sync_copy(
            buf_v.at[pl.ds(0, 16)], out_hbm.at[pl.ds(c * N + s * 640 + 384, 16)]
        )


def _mm_body(h_ref, w_ref, degp_ref, out_ref):
    deg = degp_ref[:, 0] + degp_ref[:, 1]
    norm = lax.rsqrt(jnp.where(deg > 0.0, deg, 1.0))
    acc = jnp.dot(h_ref[...], w_ref[...], preferred_element_type=jnp.float32)
    out_ref[...] = acc * norm[:, None]


def _matmul(h, W, degp):
    bm = 10000
    grid = (N // bm, NC)
    return pl.pallas_call(
        _mm_body,
        grid=grid,
        in_specs=[
            pl.BlockSpec((bm, F), lambda i, c: (i, 0)),
            pl.BlockSpec((F, FH), lambda i, c: (0, c)),
            pl.BlockSpec((bm, NC), lambda i, c: (i, 0)),
        ],
        out_specs=pl.BlockSpec((bm, FH), lambda i, c: (c * (N // bm) + i, 0)),
        out_shape=jax.ShapeDtypeStruct((NC * N, FH), jnp.float32),
    )(h, W, degp)


def _final_body(agg_ref, degp_ref, b_ref, out_ref):
    deg = degp_ref[:, 0] + degp_ref[:, 1]
    norm = lax.rsqrt(jnp.where(deg > 0.0, deg, 1.0))
    brow = jnp.where(pl.program_id(1) == 0, b_ref[0, :], b_ref[1, :])
    out_ref[...] = jnp.maximum(agg_ref[...] * norm[:, None] + brow, 0.0)


def _final(agg, degp, b2):
    bm = 1000
    nb = N // bm
    grid = (nb, NC)
    return pl.pallas_call(
        _final_body,
        grid=grid,
        in_specs=[
            pl.BlockSpec((bm, FH), lambda i, c: (c * nb + i, 0)),
            pl.BlockSpec((bm, NC), lambda i, c: (i, 0)),
            pl.BlockSpec((NC, FH), lambda i, c: (0, 0)),
        ],
        out_specs=pl.BlockSpec((bm, FH), lambda i, c: (i, c)),
        out_shape=jax.ShapeDtypeStruct((N, F), jnp.float32),
    )(agg, degp, b2)


def kernel(h, edge_index, W, b):
    ei = edge_index.astype(jnp.int32)
    src = ei[0]
    dst = ei[1]
    src2 = src.reshape(ROWS, CH)
    dst2 = dst.reshape(ROWS, CH)

    degp = _deg(dst2).reshape(NC, N).T
    hw = _matmul(h, W, degp)
    agg = _agg(hw, src2, dst2)
    return _final(agg, degp, b.reshape(NC, FH))


# speedup vs baseline: 1.0759x; 1.0533x over previous
import functools

import jax
import jax.numpy as jnp
from jax import lax
from jax.experimental import pallas as pl
from jax.experimental.pallas import tpu as pltpu
from jax.experimental.pallas import tpu_sc as plsc

N = 10000
E = 160000
F = 256
FH = F // 2
NC, NS = 2, 16
CH = 125
ROWS = E // CH
RPT_DEG = ROWS // (NC * NS)
RPT_AGG = ROWS // NS

_mesh = plsc.VectorSubcoreMesh(core_axis_name="c", subcore_axis_name="s")


@functools.partial(
    pl.kernel,
    out_type=jax.ShapeDtypeStruct((NC * N,), jnp.float32),
    mesh=_mesh,
    scratch_types=[
        pltpu.VMEM((RPT_DEG, CH), jnp.int32),
        pltpu.VMEM((128,), jnp.float32),
        pltpu.VMEM((640,), jnp.float32),
        pltpu.VMEM_SHARED((N,), jnp.float32),
    ],
)
def _deg(dst_hbm, out_hbm, didx_v, ones_v, buf_v, deg_sh):
    c = lax.axis_index("c")
    s = lax.axis_index("s")
    wid = c * NS + s

    for i in range(40):
        buf_v[pl.ds(16 * i, 16)] = jnp.zeros((16,), jnp.float32)

    @pl.when(s < NS - 1)
    def _():
        pltpu.sync_copy(buf_v.at[pl.ds(0, 640)], deg_sh.at[pl.ds(s * 640, 640)])

    @pl.when(s == NS - 1)
    def _():
        pltpu.sync_copy(buf_v.at[pl.ds(0, 400)], deg_sh.at[pl.ds(s * 640, 400)])

    for i in range(8):
        ones_v[pl.ds(16 * i, 16)] = jnp.ones((16,), jnp.float32)

    pltpu.sync_copy(dst_hbm.at[pl.ds(wid * RPT_DEG, RPT_DEG)], didx_v)
    plsc.subcore_barrier()

    def body(j, _):
        pltpu.sync_copy(ones_v.at[pl.ds(0, CH)], deg_sh.at[didx_v.at[j]], add=True)
        return 0

    lax.fori_loop(0, RPT_DEG, body, 0)
    plsc.subcore_barrier()

    @pl.when(s < NS - 1)
    def _():
        pltpu.sync_copy(deg_sh.at[pl.ds(s * 640, 640)], buf_v.at[pl.ds(0, 640)])
        pltpu.sync_copy(
            buf_v.at[pl.ds(0, 640)], out_hbm.at[pl.ds(c * N + s * 640, 640)]
        )

    @pl.when(s == NS - 1)
    def _():
        pltpu.sync_copy(deg_sh.at[pl.ds(s * 640, 400)], buf_v.at[pl.ds(0, 400)])
        pltpu.sync_copy(
            buf_v.at[pl.ds(0, 400)], out_hbm.at[pl.ds(c * N + s * 640, 400)]
        )


@functools.partial(
    pl.kernel,
    out_type=jax.ShapeDtypeStruct((NC * N, FH), jnp.float32),
    mesh=_mesh,
    scratch_types=[
        pltpu.VMEM((RPT_AGG // 2, CH), jnp.int32),
        pltpu.VMEM((RPT_AGG // 2, CH), jnp.int32),
        pltpu.VMEM((128, FH), jnp.float32),
        pltpu.VMEM((CH, FH), jnp.float32),
        pltpu.VMEM((5, 128), jnp.int32),
        pltpu.VMEM((16,), jnp.int32),
        pltpu.VMEM_SHARED((N, FH), jnp.float32),
        pltpu.SemaphoreType.DMA,
        pltpu.SemaphoreType.DMA,
    ],
)
def _agg(hw_hbm, src_hbm, dst_hbm, out_hbm, sidx_v, didx_v, buf_v, bufb_v,
         zidx_v, tidx_v, acc_sh, sema, semb):
    c = lax.axis_index("c")
    s = lax.axis_index("s")

    for j in range(5):
        for k in range(8):
            zidx_v[j, pl.ds(16 * k, 16)] = (
                s * 640 + 128 * j + 16 * k + lax.iota(jnp.int32, 16)
            )
    tidx_v[...] = s * 640 + 384 + lax.iota(jnp.int32, 16)

    def zbody(i, _):
        for k in range(FH // 16):
            buf_v[i, pl.ds(16 * k, 16)] = jnp.zeros((16,), jnp.float32)
        return 0

    lax.fori_loop(0, 128, zbody, 0)

    @pl.when(s < NS - 1)
    def _():
        for j in range(5):
            pltpu.sync_copy(buf_v, acc_sh.at[zidx_v.at[j]])

    @pl.when(s == NS - 1)
    def _():
        for j in range(3):
            pltpu.sync_copy(buf_v, acc_sh.at[zidx_v.at[j]])
        pltpu.sync_copy(buf_v.at[pl.ds(0, 16)], acc_sh.at[tidx_v])

    plsc.subcore_barrier()

    HR = RPT_AGG // 2
    bufa = buf_v.at[pl.ds(0, CH)]
    hw_c = hw_hbm.at[pl.ds(pl.multiple_of(c * N, 8), N)]
    for h in range(2):
        pltpu.sync_copy(
            src_hbm.at[pl.ds(s * RPT_AGG + h * HR, HR)], sidx_v
        )
        pltpu.sync_copy(
            dst_hbm.at[pl.ds(s * RPT_AGG + h * HR, HR)], didx_v
        )
        pltpu.async_copy(hw_c.at[sidx_v.at[0]], bufa, sema)

        def body(t, _):
            j0 = 2 * t
            db = pltpu.async_copy(hw_c.at[sidx_v.at[j0 + 1]], bufb_v, semb)
            pltpu.make_async_copy(hw_c.at[sidx_v.at[j0]], bufa, sema).wait()
            pltpu.sync_copy(bufa, acc_sh.at[didx_v.at[j0]], add=True)

            @pl.when(t < HR // 2 - 1)
            def _():
                pltpu.async_copy(hw_c.at[sidx_v.at[j0 + 2]], bufa, sema)

            db.wait()
            pltpu.sync_copy(bufb_v, acc_sh.at[didx_v.at[j0 + 1]], add=True)
            return 0

        lax.fori_loop(0, HR // 2, body, 0)
    plsc.subcore_barrier()

    @pl.when(s < NS - 1)
    def _():
        for j in range(5):
            pltpu.async_copy(acc_sh.at[zidx_v.at[j]], buf_v, sema).wait()
            pltpu.sync_copy(
                buf_v, out_hbm.at[pl.ds(c * N + s * 640 + 128 * j, 128)]
            )

    @pl.when(s == NS - 1)
    def _():
        for j in range(3):
            pltpu.async_copy(acc_sh.at[zidx_v.at[j]], buf_v, sema).wait()
            pltpu.sync_copy(
                buf_v, out_hbm.at[pl.ds(c * N + s * 640 + 128 * j, 128)]
            )
        pltpu.async_copy(acc_sh.at[tidx_v], buf_v.at[pl.ds(0, 16)], sema).wait()
        pltpu.sync_copy(
            buf_v.at[pl.ds(0, 16)], out_hbm.at[pl.ds(c * N + s * 640 + 384, 16)]
        )


def _mm_body(h_ref, w_ref, degp_ref, out_ref):
    deg = degp_ref[:, 0] + degp_ref[:, 1]
    norm = lax.rsqrt(jnp.where(deg > 0.0, deg, 1.0))
    acc = jnp.dot(h_ref[...], w_ref[...], preferred_element_type=jnp.float32)
    out_ref[...] = acc * norm[:, None]


def _matmul(h, W, degp):
    bm = 10000
    grid = (N // bm, NC)
    return pl.pallas_call(
        _mm_body,
        grid=grid,
        in_specs=[
            pl.BlockSpec((bm, F), lambda i, c: (i, 0)),
            pl.BlockSpec((F, FH), lambda i, c: (0, c)),
            pl.BlockSpec((bm, NC), lambda i, c: (i, 0)),
        ],
        out_specs=pl.BlockSpec((bm, FH), lambda i, c: (c * (N // bm) + i, 0)),
        out_shape=jax.ShapeDtypeStruct((NC * N, FH), jnp.float32),
    )(h, W, degp)


def _final_body(agg_ref, degp_ref, b_ref, out_ref):
    deg = degp_ref[:, 0] + degp_ref[:, 1]
    norm = lax.rsqrt(jnp.where(deg > 0.0, deg, 1.0))
    brow = jnp.where(pl.program_id(1) == 0, b_ref[0, :], b_ref[1, :])
    out_ref[...] = jnp.maximum(agg_ref[...] * norm[:, None] + brow, 0.0)


def _final(agg, degp, b2):
    bm = 5000
    nb = N // bm
    grid = (nb, NC)
    return pl.pallas_call(
        _final_body,
        grid=grid,
        in_specs=[
            pl.BlockSpec((bm, FH), lambda i, c: (c * nb + i, 0)),
            pl.BlockSpec((bm, NC), lambda i, c: (i, 0)),
            pl.BlockSpec((NC, FH), lambda i, c: (0, 0)),
        ],
        out_specs=pl.BlockSpec((bm, FH), lambda i, c: (i, c)),
        out_shape=jax.ShapeDtypeStruct((N, F), jnp.float32),
    )(agg, degp, b2)


def kernel(h, edge_index, W, b):
    ei = edge_index.astype(jnp.int32)
    src = ei[0]
    dst = ei[1]
    src2 = src.reshape(ROWS, CH)
    dst2 = dst.reshape(ROWS, CH)

    degp = _deg(dst2).reshape(NC, N).T
    hw = _matmul(h, W, degp)
    agg = _agg(hw, src2, dst2)
    return _final(agg, degp, b.reshape(NC, FH))


# speedup vs baseline: 1.0837x; 1.0072x over previous
import functools

import jax
import jax.numpy as jnp
from jax import lax
from jax.experimental import pallas as pl
from jax.experimental.pallas import tpu as pltpu
from jax.experimental.pallas import tpu_sc as plsc

N = 10000
E = 160000
F = 256
FH = F // 2
NC, NS = 2, 16
CH = 125
ROWS = E // CH
RPT_DEG = ROWS // (NC * NS)
RPT_AGG = ROWS // NS

_mesh = plsc.VectorSubcoreMesh(core_axis_name="c", subcore_axis_name="s")


@functools.partial(
    pl.kernel,
    out_type=jax.ShapeDtypeStruct((NC * N,), jnp.float32),
    mesh=_mesh,
    scratch_types=[
        pltpu.VMEM((RPT_DEG, CH), jnp.int32),
        pltpu.VMEM((128,), jnp.float32),
        pltpu.VMEM((640,), jnp.float32),
        pltpu.VMEM_SHARED((N,), jnp.float32),
    ],
)
def _deg(dst_hbm, out_hbm, didx_v, ones_v, buf_v, deg_sh):
    c = lax.axis_index("c")
    s = lax.axis_index("s")
    wid = c * NS + s

    for i in range(40):
        buf_v[pl.ds(16 * i, 16)] = jnp.zeros((16,), jnp.float32)

    @pl.when(s < NS - 1)
    def _():
        pltpu.sync_copy(buf_v.at[pl.ds(0, 640)], deg_sh.at[pl.ds(s * 640, 640)])

    @pl.when(s == NS - 1)
    def _():
        pltpu.sync_copy(buf_v.at[pl.ds(0, 400)], deg_sh.at[pl.ds(s * 640, 400)])

    for i in range(8):
        ones_v[pl.ds(16 * i, 16)] = jnp.ones((16,), jnp.float32)

    pltpu.sync_copy(dst_hbm.at[pl.ds(wid * RPT_DEG, RPT_DEG)], didx_v)
    plsc.subcore_barrier()

    def body(j, _):
        pltpu.sync_copy(ones_v.at[pl.ds(0, CH)], deg_sh.at[didx_v.at[j]], add=True)
        return 0

    lax.fori_loop(0, RPT_DEG, body, 0)
    plsc.subcore_barrier()

    @pl.when(s < NS - 1)
    def _():
        pltpu.sync_copy(deg_sh.at[pl.ds(s * 640, 640)], buf_v.at[pl.ds(0, 640)])
        pltpu.sync_copy(
            buf_v.at[pl.ds(0, 640)], out_hbm.at[pl.ds(c * N + s * 640, 640)]
        )

    @pl.when(s == NS - 1)
    def _():
        pltpu.sync_copy(deg_sh.at[pl.ds(s * 640, 400)], buf_v.at[pl.ds(0, 400)])
        pltpu.sync_copy(
            buf_v.at[pl.ds(0, 400)], out_hbm.at[pl.ds(c * N + s * 640, 400)]
        )


@functools.partial(
    pl.kernel,
    out_type=jax.ShapeDtypeStruct((NC * N, FH), jnp.float32),
    mesh=_mesh,
    scratch_types=[
        pltpu.VMEM((RPT_AGG // 2, CH), jnp.int32),
        pltpu.VMEM((RPT_AGG // 2, CH), jnp.int32),
        pltpu.VMEM((128, FH), jnp.float32),
        pltpu.VMEM((CH, FH), jnp.float32),
        pltpu.VMEM((5, 128), jnp.int32),
        pltpu.VMEM((16,), jnp.int32),
        pltpu.VMEM_SHARED((N, FH), jnp.float32),
        pltpu.SemaphoreType.DMA,
        pltpu.SemaphoreType.DMA,
    ],
)
def _agg(hw_hbm, src_hbm, dst_hbm, out_hbm, sidx_v, didx_v, buf_v, bufb_v,
         zidx_v, tidx_v, acc_sh, sema, semb):
    c = lax.axis_index("c")
    s = lax.axis_index("s")

    for j in range(5):
        for k in range(8):
            zidx_v[j, pl.ds(16 * k, 16)] = (
                s * 640 + 128 * j + 16 * k + lax.iota(jnp.int32, 16)
            )
    tidx_v[...] = s * 640 + 384 + lax.iota(jnp.int32, 16)

    def zbody(i, _):
        for k in range(FH // 16):
            buf_v[i, pl.ds(16 * k, 16)] = jnp.zeros((16,), jnp.float32)
        return 0

    lax.fori_loop(0, 128, zbody, 0)

    @pl.when(s < NS - 1)
    def _():
        for j in range(5):
            pltpu.sync_copy(buf_v, acc_sh.at[zidx_v.at[j]])

    @pl.when(s == NS - 1)
    def _():
        for j in range(3):
            pltpu.sync_copy(buf_v, acc_sh.at[zidx_v.at[j]])
        pltpu.sync_copy(buf_v.at[pl.ds(0, 16)], acc_sh.at[tidx_v])

    plsc.subcore_barrier()

    HR = RPT_AGG // 2
    bufa = buf_v.at[pl.ds(0, CH)]
    hw_c = hw_hbm.at[pl.ds(pl.multiple_of(c * N, 8), N)]
    for h in range(2):
        pltpu.sync_copy(
            src_hbm.at[pl.ds(s * RPT_AGG + h * HR, HR)], sidx_v
        )
        pltpu.sync_copy(
            dst_hbm.at[pl.ds(s * RPT_AGG + h * HR, HR)], didx_v
        )
        pltpu.async_copy(hw_c.at[sidx_v.at[0]], bufa, sema)

        def body(t, _):
            j0 = 2 * t
            db = pltpu.async_copy(hw_c.at[sidx_v.at[j0 + 1]], bufb_v, semb)
            pltpu.make_async_copy(hw_c.at[sidx_v.at[j0]], bufa, sema).wait()
            pltpu.sync_copy(bufa, acc_sh.at[didx_v.at[j0]], add=True)

            @pl.when(t < HR // 2 - 1)
            def _():
                pltpu.async_copy(hw_c.at[sidx_v.at[j0 + 2]], bufa, sema)

            db.wait()
            pltpu.sync_copy(bufb_v, acc_sh.at[didx_v.at[j0 + 1]], add=True)
            return 0

        lax.fori_loop(0, HR // 2, body, 0)
    plsc.subcore_barrier()

    @pl.when(s < NS - 1)
    def _():
        for j in range(5):
            pltpu.async_copy(acc_sh.at[zidx_v.at[j]], buf_v, sema).wait()
            pltpu.sync_copy(
                buf_v, out_hbm.at[pl.ds(c * N + s * 640 + 128 * j, 128)]
            )

    @pl.when(s == NS - 1)
    def _():
        for j in range(3):
            pltpu.async_copy(acc_sh.at[zidx_v.at[j]], buf_v, sema).wait()
            pltpu.sync_copy(
                buf_v, out_hbm.at[pl.ds(c * N + s * 640 + 128 * j, 128)]
            )
        pltpu.async_copy(acc_sh.at[tidx_v], buf_v.at[pl.ds(0, 16)], sema).wait()
        pltpu.sync_copy(
            buf_v.at[pl.ds(0, 16)], out_hbm.at[pl.ds(c * N + s * 640 + 384, 16)]
        )


def _mm_body(h_ref, w_ref, degp_ref, out_ref):
    deg = degp_ref[:, 0] + degp_ref[:, 1]
    norm = lax.rsqrt(jnp.where(deg > 0.0, deg, 1.0))
    acc = jnp.dot(h_ref[...], w_ref[...], preferred_element_type=jnp.float32)
    out_ref[...] = acc * norm[:, None]


def _matmul(h, W, degp):
    bm = 10000
    grid = (N // bm, NC)
    return pl.pallas_call(
        _mm_body,
        grid=grid,
        in_specs=[
            pl.BlockSpec((bm, F), lambda i, c: (i, 0)),
            pl.BlockSpec((F, FH), lambda i, c: (0, c)),
            pl.BlockSpec((bm, NC), lambda i, c: (i, 0)),
        ],
        out_specs=pl.BlockSpec((bm, FH), lambda i, c: (c * (N // bm) + i, 0)),
        out_shape=jax.ShapeDtypeStruct((NC * N, FH), jnp.float32),
    )(h, W, degp)


def _final_body(agg_ref, degp_ref, b_ref, out_ref):
    deg = degp_ref[:, 0] + degp_ref[:, 1]
    norm = lax.rsqrt(jnp.where(deg > 0.0, deg, 1.0))
    brow = jnp.where(pl.program_id(1) == 0, b_ref[0, :], b_ref[1, :])
    out_ref[...] = jnp.maximum(agg_ref[...] * norm[:, None] + brow, 0.0)


def _final(agg, degp, b2):
    bm = 10000
    nb = N // bm
    grid = (nb, NC)
    return pl.pallas_call(
        _final_body,
        grid=grid,
        in_specs=[
            pl.BlockSpec((bm, FH), lambda i, c: (c * nb + i, 0)),
            pl.BlockSpec((bm, NC), lambda i, c: (i, 0)),
            pl.BlockSpec((NC, FH), lambda i, c: (0, 0)),
        ],
        out_specs=pl.BlockSpec((bm, FH), lambda i, c: (i, c)),
        out_shape=jax.ShapeDtypeStruct((N, F), jnp.float32),
    )(agg, degp, b2)


def kernel(h, edge_index, W, b):
    ei = edge_index.astype(jnp.int32)
    src = ei[0]
    dst = ei[1]
    src2 = src.reshape(ROWS, CH)
    dst2 = dst.reshape(ROWS, CH)

    degp = _deg(dst2).reshape(NC, N).T
    hw = _matmul(h, W, degp)
    agg = _agg(hw, src2, dst2)
    return _final(agg, degp, b.reshape(NC, FH))
